# baseline (speedup 1.0000x reference)
import jax
import jax.numpy as jnp
from jax.experimental import pallas as pl
from jax.experimental.pallas import tpu as pltpu

_H = 32
_P = 8
_T_ENC = 16
_PACK = 4
_BLOCK = 5000


def _body(t_ref, pa_ref, wt1_ref, bt1_ref, wt2_ref, bt2_ref,
          wd1pt_ref, bd1_ref, w1p_ref, w2p_ref, w3p_ref,
          b2p_ref, b3p_ref,
          ze_ref, out_ref, const_ref):
    @pl.when(pl.program_id(0) == 0)
    def _():
        tv = jnp.dot(
            jnp.maximum(jnp.dot(t_ref[...], wt1_ref[...],
                                preferred_element_type=jnp.float32)
                        + bt1_ref[...], 0.0),
            wt2_ref[...], preferred_element_type=jnp.float32) + bt2_ref[...]
        const_ref[...] = (
            jnp.dot(pa_ref[...], wd1pt_ref[:_P, :],
                    preferred_element_type=jnp.float32)
            + jnp.dot(tv, wd1pt_ref[_P:, :],
                      preferred_element_type=jnp.float32)
            + bd1_ref[...])

    z = ze_ref[...].astype(jnp.float32)
    h = jnp.maximum(
        jnp.dot(z, w1p_ref[...], preferred_element_type=jnp.float32)
        + const_ref[...], 0.0)
    h = jnp.maximum(
        jnp.dot(h, w2p_ref[...], preferred_element_type=jnp.float32)
        + b2p_ref[...], 0.0)
    v = (jnp.dot(h, w3p_ref[...], preferred_element_type=jnp.float32)
         + b3p_ref[...])
    out_ref[...] = v.astype(jnp.bfloat16)


def kernel(t, zone_embedding, zone_features, edge_index, person_attrs,
           W1, b1, W2, b2, Wt1, bt1, Wt2, bt2,
           Wd1, bd1, Wd2, bd2, Wd3, bd3):
    del zone_features, edge_index, W1, b1, W2, b2
    n = zone_embedding.shape[0]
    n4 = n // _PACK
    eye = jnp.eye(_PACK, dtype=jnp.float32)

    ze4 = (jnp.reshape(zone_embedding, (n4, _PACK, _H))
           .swapaxes(1, 2)
           .reshape(n4, _PACK * _H)
           .astype(jnp.bfloat16))

    ones_row = jnp.ones((1, _PACK), jnp.float32)
    w1p = jnp.kron(Wd1[:_H], eye)
    w2p = jnp.kron(Wd2, eye)
    w3p = jnp.kron(Wd3, eye)
    wd1pt_i = jnp.kron(Wd1[_H:], ones_row)
    bd1_i = jnp.reshape(
        jnp.broadcast_to(bd1[:, None], (2 * _H, _PACK)), (1, _PACK * 2 * _H))
    b2p = jnp.reshape(
        jnp.broadcast_to(bd2[:, None], (_H, _PACK)), (1, _PACK * _H))
    b3p = jnp.reshape(
        jnp.broadcast_to(bd3[:, None], (_H, _PACK)), (1, _PACK * _H))

    grid = (n4 // _BLOCK,)

    def full(shape):
        return pl.BlockSpec(shape, lambda i: (0,) * len(shape))

    out = pl.pallas_call(
        _body,
        grid=grid,
        in_specs=[
            full((1, 1)),
            full((1, _P)),
            full(Wt1.shape),
            full((1, _T_ENC)),
            full(Wt2.shape),
            full((1, _T_ENC)),
            full((_P + _T_ENC, _PACK * 2 * _H)),
            full((1, _PACK * 2 * _H)),
            full((_PACK * _H, _PACK * 2 * _H)),
            full((_PACK * 2 * _H, _PACK * _H)),
            full((_PACK * _H, _PACK * _H)),
            full((1, _PACK * _H)),
            full((1, _PACK * _H)),
            pl.BlockSpec((_BLOCK, _PACK * _H), lambda i: (i, 0)),
        ],
        out_specs=pl.BlockSpec((_BLOCK, _PACK * _H), lambda i: (i, 0)),
        out_shape=jax.ShapeDtypeStruct((n4, _PACK * _H), jnp.bfloat16),
        scratch_shapes=[pltpu.VMEM((1, _PACK * 2 * _H), jnp.float32)],
    )(
        jnp.reshape(t, (1, 1)),
        jnp.reshape(person_attrs, (1, _P)),
        Wt1,
        jnp.reshape(bt1, (1, _T_ENC)),
        Wt2,
        jnp.reshape(bt2, (1, _T_ENC)),
        wd1pt_i,
        bd1_i,
        w1p,
        w2p,
        w3p,
        b2p,
        b3p,
        ze4,
    )
    return (out.astype(jnp.float32)
            .reshape(n4, _H, _PACK)
            .swapaxes(1, 2)
            .reshape(n, _H))


# speedup vs baseline: 1.8206x; 1.8206x over previous
import jax
import jax.numpy as jnp
from jax.experimental import pallas as pl
from jax.experimental.pallas import tpu as pltpu

_H = 32
_P = 8
_T_ENC = 16
_BLOCK = 20000


def _body(t_ref, pa_ref, wt1_ref, bt1_ref, wt2_ref, bt2_ref,
          wd1_ref, bd1_ref, wd2_ref, bd2_ref, wd3_ref, bd3_ref,
          ze_ref, out_ref, const_ref):
    @pl.when(pl.program_id(0) == 0)
    def _():
        tv = jnp.dot(
            jnp.maximum(jnp.dot(t_ref[...], wt1_ref[...],
                                preferred_element_type=jnp.float32)
                        + bt1_ref[...], 0.0),
            wt2_ref[...], preferred_element_type=jnp.float32) + bt2_ref[...]
        wd1 = wd1_ref[...]
        const_ref[...] = (
            jnp.dot(pa_ref[...], wd1[_H:_H + _P, :],
                    preferred_element_type=jnp.float32)
            + jnp.dot(tv, wd1[_H + _P:, :], preferred_element_type=jnp.float32)
            + bd1_ref[...])

    z = ze_ref[...].astype(jnp.float32)
    h = jnp.maximum(
        jnp.dot(z, wd1_ref[:_H, :], preferred_element_type=jnp.float32)
        + const_ref[...], 0.0)
    h = jnp.maximum(
        jnp.dot(h, wd2_ref[...], preferred_element_type=jnp.float32)
        + bd2_ref[...], 0.0)
    v = (jnp.dot(h, wd3_ref[...], preferred_element_type=jnp.float32)
         + bd3_ref[...])
    out_ref[...] = v.astype(jnp.bfloat16)


def kernel(t, zone_embedding, zone_features, edge_index, person_attrs,
           W1, b1, W2, b2, Wt1, bt1, Wt2, bt2,
           Wd1, bd1, Wd2, bd2, Wd3, bd3):
    del zone_features, edge_index, W1, b1, W2, b2
    n = zone_embedding.shape[0]
    grid = (n // _BLOCK,)

    def full(shape):
        return pl.BlockSpec(shape, lambda i: (0,) * len(shape))

    out = pl.pallas_call(
        _body,
        grid=grid,
        in_specs=[
            full((1, 1)),
            full((1, _P)),
            full(Wt1.shape),
            full((1, _T_ENC)),
            full(Wt2.shape),
            full((1, _T_ENC)),
            full(Wd1.shape),
            full((1, 2 * _H)),
            full(Wd2.shape),
            full((1, _H)),
            full(Wd3.shape),
            full((1, _H)),
            pl.BlockSpec((_BLOCK, _H), lambda i: (i, 0)),
        ],
        out_specs=pl.BlockSpec((_BLOCK, _H), lambda i: (i, 0)),
        out_shape=jax.ShapeDtypeStruct((n, _H), jnp.bfloat16),
        scratch_shapes=[pltpu.VMEM((1, 2 * _H), jnp.float32)],
    )(
        jnp.reshape(t, (1, 1)),
        jnp.reshape(person_attrs, (1, _P)),
        Wt1,
        jnp.reshape(bt1, (1, _T_ENC)),
        Wt2,
        jnp.reshape(bt2, (1, _T_ENC)),
        Wd1,
        jnp.reshape(bd1, (1, 2 * _H)),
        Wd2,
        jnp.reshape(bd2, (1, _H)),
        Wd3,
        jnp.reshape(bd3, (1, _H)),
        zone_embedding.astype(jnp.bfloat16),
    )
    return out.astype(jnp.float32)
